# baseline (speedup 1.0000x reference)
import functools

import jax
import jax.numpy as jnp
from jax import lax
from jax.experimental import pallas as pl
from jax.experimental.pallas import tpu as pltpu
from jax.experimental.pallas import tpu_sc as plsc

DIM = 64
CHUNK = 400
NROW = 4
NIDX = 4


@functools.cache
def _make_detile(vocab: int, dim: int):
    info = plsc.get_sparse_core_info()
    nw = info.num_cores * info.num_subcores
    n_cols_full = vocab // 128
    tail = vocab - n_cols_full * 128
    assert dim == 64 and tail % 8 == 0

    mesh = plsc.VectorSubcoreMesh(core_axis_name="c", subcore_axis_name="s")
    NB = 2

    @functools.partial(
        pl.kernel,
        mesh=mesh,
        out_type=jax.ShapeDtypeStruct((vocab, 2 * dim), jnp.float32),
        scratch_types=[
            pltpu.VMEM((NB, dim, 128), jnp.float32),
            pltpu.VMEM((NB, 128, 128), jnp.float32),
            pltpu.SemaphoreType.DMA,
            pltpu.SemaphoreType.DMA,
            pltpu.SemaphoreType.DMA,
            pltpu.SemaphoreType.DMA,
        ],
        compiler_params=pltpu.CompilerParams(
            use_tc_tiling_on_sc=True, needs_layout_passes=False),
    )
    def detile_kernel(tab_t_hbm, tail_hbm, out_hbm, tin, tout, ls0, ls1,
                      ws0, ws1):
        lsems = (ls0, ls1)
        wsems = (ws0, ws1)
        wid = lax.axis_index("s") * info.num_cores + lax.axis_index("c")
        per_w = 2 * ((n_cols_full + 2 * nw - 1) // (2 * nw))
        lo = wid * per_w
        hi = jnp.minimum(lo + per_w, n_cols_full)

        def load_start(tc, s):
            pltpu.async_copy(
                tab_t_hbm.at[pl.ds(0, dim), pl.ds(tc * 128, 128)],
                tin.at[s], lsems[s])

        def load_wait(s):
            pltpu.make_async_copy(
                tab_t_hbm.at[pl.ds(0, dim), pl.ds(0, 128)], tin.at[s],
                lsems[s]).wait()

        def write_start(tc, s):
            pltpu.async_copy(
                tout.at[s], out_hbm.at[pl.ds(tc * 128, 128)], wsems[s])

        def write_wait(s):
            pltpu.make_async_copy(
                tout.at[s], out_hbm.at[pl.ds(0, 128)], wsems[s]).wait()

        d_base = jax.lax.broadcasted_iota(jnp.int32, (16,), 0)

        dvecs = [d_base + 16 * q for q in range(dim // 16)]

        def transpose(s):
            pend = []
            for q in range(dim // 16):
                for v in range(128):
                    col = plsc.load_gather(
                        tin.at[s], [dvecs[q], jnp.full((16,), v, jnp.int32)])
                    pend.append((q, v, col))
                    if len(pend) >= 6:
                        qq, vv, cc = pend.pop(0)
                        tout[s, vv, pl.ds(16 * qq, 16)] = cc
            for qq, vv, cc in pend:
                tout[s, vv, pl.ds(16 * qq, 16)] = cc

        n_my = hi - lo
        n_pairs = n_my // 2

        @pl.when(n_pairs > 0)
        def _():
            load_start(lo, 0)

            def body(o, carry):
                tc0 = lo + 2 * o
                load_start(tc0 + 1, 1)
                load_wait(0)
                @pl.when(o >= 1)
                def _():
                    write_wait(0)
                transpose(0)
                write_start(tc0, 0)
                @pl.when(o + 1 < n_pairs)
                def _():
                    load_start(tc0 + 2, 0)
                load_wait(1)
                @pl.when(o >= 1)
                def _():
                    write_wait(1)
                transpose(1)
                write_start(tc0 + 1, 1)
                return carry

            lax.fori_loop(0, n_pairs, body, 0)
            write_wait(0)
            write_wait(1)

        if tail:
            @pl.when(wid == nw - 1)
            def _():
                pltpu.sync_copy(
                    tail_hbm, out_hbm.at[pl.ds(n_cols_full * 128, tail)])

    return detile_kernel


@functools.cache
def _make_gather(n_total: int, dim: int):
    info = plsc.get_sparse_core_info()
    nw = info.num_cores * info.num_subcores
    per_w = n_total // nw
    n_chunks = per_w // CHUNK
    assert per_w * nw == n_total and n_chunks * CHUNK == per_w
    assert n_chunks % NIDX == 0 and n_chunks // NIDX >= 2

    mesh = plsc.VectorSubcoreMesh(core_axis_name="c", subcore_axis_name="s")

    @functools.partial(
        pl.kernel,
        mesh=mesh,
        out_type=jax.ShapeDtypeStruct((n_total, 2 * dim), jnp.float32),
        scratch_types=[
            pltpu.VMEM((NIDX, CHUNK), jnp.int32),
            pltpu.VMEM((NROW, CHUNK, dim), jnp.float32),
            pltpu.SemaphoreType.DMA,
            pltpu.SemaphoreType.DMA,
            pltpu.SemaphoreType.DMA,
            pltpu.SemaphoreType.DMA,
            pltpu.SemaphoreType.DMA,
            pltpu.SemaphoreType.DMA,
            pltpu.SemaphoreType.DMA,
            pltpu.SemaphoreType.DMA,
            pltpu.SemaphoreType.DMA,
            pltpu.SemaphoreType.DMA,
            pltpu.SemaphoreType.DMA,
            pltpu.SemaphoreType.DMA,
        ],
        compiler_params=pltpu.CompilerParams(use_tc_tiling_on_sc=False),
    )
    def gather_kernel(idx_hbm, table_hbm, out_hbm, idx_v, rows_v,
                      isem0, isem1, isem2, isem3,
                      gsem0, gsem1, gsem2, gsem3,
                      osem0, osem1, osem2, osem3):
        isems = (isem0, isem1, isem2, isem3)
        gsems = (gsem0, gsem1, gsem2, gsem3)
        osems = (osem0, osem1, osem2, osem3)
        wid = lax.axis_index("s") * info.num_cores + lax.axis_index("c")
        base = wid * per_w

        def idx_load_start(c, j):
            pltpu.async_copy(
                idx_hbm.at[pl.ds(base + c * CHUNK, CHUNK)], idx_v.at[j], isems[j])

        def idx_load_wait(j):
            pltpu.make_async_copy(
                idx_hbm.at[pl.ds(base, CHUNK)], idx_v.at[j], isems[j]).wait()

        def gather_wait(b):
            pltpu.make_async_copy(
                table_hbm.at[idx_v.at[b]], rows_v.at[b], gsems[b]).wait()

        def write_start(c, b):
            pltpu.async_copy(
                rows_v.at[b],
                out_hbm.at[pl.ds(base + c * CHUNK, CHUNK), pl.ds(0, dim)],
                osems[b])

        def out_write_wait(b):
            pltpu.make_async_copy(
                rows_v.at[b],
                out_hbm.at[pl.ds(base, CHUNK), pl.ds(0, dim)], osems[b]).wait()

        def visit(c, k, first=False, prefetch=True):
            b = k % NROW
            if not first:
                out_write_wait(b)
            idx_load_wait(b)
            pltpu.async_copy(table_hbm.at[idx_v.at[b]], rows_v.at[b], gsems[b])
            if prefetch:
                idx_load_start(c + 2, (k + 2) % NIDX)
            kp = (k - 1) % NROW
            if not (first and k == 0):
                gather_wait(kp)
                write_start(c - 1, kp)

        n_quads = n_chunks // NIDX

        idx_load_start(0, 0)
        idx_load_start(1, 1)
        for k in range(NIDX):
            visit(k, k, first=True)

        def quad(o, carry):
            cb = o * NIDX
            for k in range(NIDX):
                visit(cb + k, k)
            return carry

        lax.fori_loop(1, n_quads - 1, quad, 0)

        cb = (n_quads - 1) * NIDX
        for k in range(NIDX):
            visit(cb + k, k, prefetch=(k < 2))

        last = n_chunks - 1
        bl = last % NROW
        gather_wait(bl)
        write_start(last, bl)
        for b in range(NROW):
            out_write_wait(b)

    return gather_kernel


def kernel(x, table):
    b, l = x.shape
    n = b * l
    vocab = table.shape[0]
    flat2 = x.reshape(n).astype(jnp.int32) * 2
    n_cols_full = vocab // 128
    tail = vocab - n_cols_full * 128
    tail_block = jnp.concatenate(
        [table[n_cols_full * 128:],
         jnp.zeros((tail, DIM), jnp.float32)], axis=1)
    pairs128 = _make_detile(vocab, DIM)(table.T, tail_block)
    pairs = pairs128.reshape(2 * vocab, DIM)
    out_pad = _make_gather(n, DIM)(flat2, pairs)
    return out_pad[:, :DIM].reshape(b, l, DIM)


# speedup vs baseline: 1.3115x; 1.3115x over previous
import functools

import jax
import jax.numpy as jnp
from jax import lax
from jax.experimental import pallas as pl
from jax.experimental.pallas import tpu as pltpu
from jax.experimental.pallas import tpu_sc as plsc

DIM = 64
CHUNK = 400
NROW = 4
NIDX = 4


@functools.cache
def _make_gather(n_total: int, dim: int):
    info = plsc.get_sparse_core_info()
    nw = info.num_cores * info.num_subcores
    per_w = n_total // nw
    n_chunks = per_w // CHUNK
    assert per_w * nw == n_total and n_chunks * CHUNK == per_w
    assert n_chunks % NIDX == 0 and n_chunks // NIDX >= 2

    mesh = plsc.VectorSubcoreMesh(core_axis_name="c", subcore_axis_name="s")

    @functools.partial(
        pl.kernel,
        mesh=mesh,
        out_type=jax.ShapeDtypeStruct((n_total, 2 * dim), jnp.float32),
        scratch_types=[
            pltpu.VMEM((NIDX, CHUNK), jnp.int32),
            pltpu.VMEM((NROW, CHUNK, dim), jnp.float32),
            pltpu.SemaphoreType.DMA,
            pltpu.SemaphoreType.DMA,
            pltpu.SemaphoreType.DMA,
            pltpu.SemaphoreType.DMA,
            pltpu.SemaphoreType.DMA,
            pltpu.SemaphoreType.DMA,
            pltpu.SemaphoreType.DMA,
            pltpu.SemaphoreType.DMA,
            pltpu.SemaphoreType.DMA,
            pltpu.SemaphoreType.DMA,
            pltpu.SemaphoreType.DMA,
            pltpu.SemaphoreType.DMA,
        ],
        compiler_params=pltpu.CompilerParams(use_tc_tiling_on_sc=False),
    )
    def gather_kernel(idx_hbm, table_hbm, out_hbm, idx_v, rows_v,
                      isem0, isem1, isem2, isem3,
                      gsem0, gsem1, gsem2, gsem3,
                      osem0, osem1, osem2, osem3):
        isems = (isem0, isem1, isem2, isem3)
        gsems = (gsem0, gsem1, gsem2, gsem3)
        osems = (osem0, osem1, osem2, osem3)
        wid = lax.axis_index("s") * info.num_cores + lax.axis_index("c")
        base = wid * per_w

        def idx_load_start(c, j):
            pltpu.async_copy(
                idx_hbm.at[pl.ds(base + c * CHUNK, CHUNK)], idx_v.at[j], isems[j])

        def idx_load_wait(j):
            pltpu.make_async_copy(
                idx_hbm.at[pl.ds(base, CHUNK)], idx_v.at[j], isems[j]).wait()

        def gather_wait(b):
            pltpu.make_async_copy(
                table_hbm.at[idx_v.at[b]], rows_v.at[b], gsems[b]).wait()

        def write_start(c, b):
            pltpu.async_copy(
                rows_v.at[b],
                out_hbm.at[pl.ds(base + c * CHUNK, CHUNK), pl.ds(0, dim)],
                osems[b])

        def out_write_wait(b):
            pltpu.make_async_copy(
                rows_v.at[b],
                out_hbm.at[pl.ds(base, CHUNK), pl.ds(0, dim)], osems[b]).wait()

        def visit(c, k, first=False, prefetch=True):
            b = k % NROW
            if not first:
                out_write_wait(b)
            idx_load_wait(b)
            pltpu.async_copy(table_hbm.at[idx_v.at[b]], rows_v.at[b], gsems[b])
            if prefetch:
                idx_load_start(c + 2, (k + 2) % NIDX)
            kp = (k - 1) % NROW
            if not (first and k == 0):
                gather_wait(kp)
                write_start(c - 1, kp)

        n_quads = n_chunks // NIDX

        idx_load_start(0, 0)
        idx_load_start(1, 1)
        for k in range(NIDX):
            visit(k, k, first=True)

        def quad(o, carry):
            cb = o * NIDX
            for k in range(NIDX):
                visit(cb + k, k)
            return carry

        lax.fori_loop(1, n_quads - 1, quad, 0)

        cb = (n_quads - 1) * NIDX
        for k in range(NIDX):
            visit(cb + k, k, prefetch=(k < 2))

        last = n_chunks - 1
        bl = last % NROW
        gather_wait(bl)
        write_start(last, bl)
        for b in range(NROW):
            out_write_wait(b)

    return gather_kernel


def kernel(x, table):
    b, l = x.shape
    n = b * l
    flat = x.reshape(n).astype(jnp.int32)
    out_pad = _make_gather(n, DIM)(flat, table)
    return out_pad[:, :DIM].reshape(b, l, DIM)
